# Initial kernel scaffold; baseline (speedup 1.0000x reference)
import jax
import jax.numpy as jnp
from jax.experimental import pallas as pl


def kernel(x, edge_index, W1_rel, W1_root, b1, W2_rel, W2_root, b2):
    raise NotImplementedError("write your pallas kernel here")


# speedup vs baseline: 5.1096x; 5.1096x over previous
import functools

import jax
import jax.numpy as jnp
from jax import lax
from jax.experimental import pallas as pl
from jax.experimental.pallas import tpu as pltpu
from jax.experimental.pallas import tpu_sc as plsc

N_NODES = 10000
N_PAD = 10112
D = 128
N_EDGES = 320000

NC = 2
NS = 16
NW = NC * NS

EDGES_PER_W = N_EDGES // NW
CHUNK = 80
N_CHUNKS = EDGES_PER_W // CHUNK
ROWS_PER_S = N_PAD // NS

_sc_mesh = plsc.VectorSubcoreMesh(
    core_axis_name="c", subcore_axis_name="s", num_cores=NC, num_subcores=NS
)


@functools.partial(
    pl.kernel,
    out_type=jax.ShapeDtypeStruct((NC * N_PAD, D), jnp.float32),
    mesh=_sc_mesh,
    scratch_types=[
        pltpu.VMEM((CHUNK,), jnp.int32),
        pltpu.VMEM((CHUNK,), jnp.int32),
        pltpu.VMEM((CHUNK, D), jnp.float32),
        pltpu.VMEM_SHARED((N_PAD, D), jnp.float32),
        pltpu.SemaphoreType.DMA,
    ],
)
def _sc_agg(x_hbm, src_hbm, dst_hbm, zeros_hbm, out_hbm,
            src_v, dst_v, rows_v, agg_sh, sem):
    c = lax.axis_index("c")
    s = lax.axis_index("s")
    wid = c * NS + s

    row0 = s * ROWS_PER_S
    pltpu.sync_copy(zeros_hbm.at[pl.ds(row0, ROWS_PER_S)],
                    agg_sh.at[pl.ds(row0, ROWS_PER_S)])
    plsc.subcore_barrier()

    base = wid * EDGES_PER_W

    def body(i, _):
        off = base + i * CHUNK
        pltpu.sync_copy(src_hbm.at[pl.ds(off, CHUNK)], src_v)
        pltpu.sync_copy(dst_hbm.at[pl.ds(off, CHUNK)], dst_v)
        pltpu.async_copy(x_hbm.at[src_v], rows_v, sem).wait()
        pltpu.sync_copy(rows_v, agg_sh.at[dst_v], add=True)
        return _

    lax.fori_loop(0, N_CHUNKS, body, 0)
    plsc.subcore_barrier()

    out_row = c * N_PAD + row0
    pltpu.sync_copy(agg_sh.at[pl.ds(row0, ROWS_PER_S)],
                    out_hbm.at[pl.ds(out_row, ROWS_PER_S)])


BR = 1264


def _dense_body(p_ref, x_ref, wrel_ref, wroot_ref, b_ref, o_ref, *, relu):
    agg = p_ref[0] + p_ref[1]
    acc = jnp.dot(agg, wrel_ref[...], preferred_element_type=jnp.float32)
    acc += jnp.dot(x_ref[...], wroot_ref[...], preferred_element_type=jnp.float32)
    acc += b_ref[...]
    if relu:
        acc = jnp.maximum(acc, 0.0)
    o_ref[...] = acc


def _dense(partials, x, w_rel, w_root, b, relu):
    p3 = partials.reshape(NC, N_PAD, D)
    return pl.pallas_call(
        functools.partial(_dense_body, relu=relu),
        grid=(N_PAD // BR,),
        in_specs=[
            pl.BlockSpec((NC, BR, D), lambda i: (0, i, 0)),
            pl.BlockSpec((BR, D), lambda i: (i, 0)),
            pl.BlockSpec((D, D), lambda i: (0, 0)),
            pl.BlockSpec((D, D), lambda i: (0, 0)),
            pl.BlockSpec((1, D), lambda i: (0, 0)),
        ],
        out_specs=pl.BlockSpec((BR, D), lambda i: (i, 0)),
        out_shape=jax.ShapeDtypeStruct((N_PAD, D), jnp.float32),
    )(p3, x, w_rel, w_root, b.reshape(1, D))


def kernel(x, edge_index, W1_rel, W1_root, b1, W2_rel, W2_root, b2):
    src = edge_index[0].astype(jnp.int32)
    dst = edge_index[1].astype(jnp.int32)
    zeros = jnp.zeros((N_PAD, D), jnp.float32)
    x_pad = jnp.concatenate([x, jnp.zeros((N_PAD - N_NODES, D), jnp.float32)])

    p1 = _sc_agg(x_pad, src, dst, zeros)
    h = _dense(p1, x_pad, W1_rel, W1_root, b1, relu=True)
    p2 = _sc_agg(h, src, dst, zeros)
    out = _dense(p2, h, W2_rel, W2_root, b2, relu=False)
    return out[:N_NODES]
